# Initial kernel scaffold; baseline (speedup 1.0000x reference)
import jax
import jax.numpy as jnp
from jax.experimental import pallas as pl


def kernel(x, Wg, Wgu, bgu, Wd, bd):
    raise NotImplementedError("write your pallas kernel here")


# speedup vs baseline: 1.2232x; 1.2232x over previous
import functools

import jax
import jax.numpy as jnp
from jax import lax
from jax.experimental import pallas as pl
from jax.experimental.pallas import tpu as pltpu


def _gate_weights(xt, wg, e, n_experts):
    logits = jax.lax.dot_general(
        xt, wg, (((1,), (1,)), ((), ())),
        preferred_element_type=jnp.float32)
    idx = jax.lax.broadcasted_iota(jnp.int32, logits.shape, 1)
    m1 = jnp.max(logits, axis=-1, keepdims=True)
    i1 = jnp.min(jnp.where(logits == m1, idx, n_experts), axis=-1,
                 keepdims=True)
    l2 = jnp.where(idx == i1, -jnp.inf, logits)
    m2 = jnp.max(l2, axis=-1, keepdims=True)
    i2 = jnp.min(jnp.where(l2 == m2, idx, n_experts), axis=-1, keepdims=True)
    dexp = jnp.exp(m2 - m1)
    s = 1.0 / (1.0 + dexp)
    return jnp.where(i1 == e, s, 0.0) + jnp.where(i2 == e, dexp * s, 0.0)


def _dense_moe_body(x_ref, wg_ref, wgug_ref, wguu_ref, bgug_ref, bguu_ref,
                    wd_ref, bd_ref, out_ref, acc_ref, *, bt, d, nj, n_experts):
    e = pl.program_id(0)
    j = pl.program_id(1)
    t = pl.program_id(2)
    xt = x_ref[...]

    we = _gate_weights(xt, wg_ref[...], e, n_experts)

    g = jax.lax.dot_general(
        xt, wgug_ref[0], (((1,), (1,)), ((), ())),
        preferred_element_type=jnp.float32) + bgug_ref[0]
    u = jax.lax.dot_general(
        xt, wguu_ref[0], (((1,), (1,)), ((), ())),
        preferred_element_type=jnp.float32) + bguu_ref[0]
    h = g * jax.nn.sigmoid(g) * u
    oe = jax.lax.dot_general(
        h, wd_ref[0], (((1,), (1,)), ((), ())),
        preferred_element_type=jnp.float32)
    oe = jnp.where(j == 0, oe + bd_ref[0], oe)
    contrib = we * oe

    sl = pl.ds(t * bt, bt)

    @pl.when(jnp.logical_and(e == 0, j == 0))
    def _():
        acc_ref[sl, :] = jnp.zeros((bt, d), jnp.float32)

    new_acc = acc_ref[sl, :] + contrib
    acc_ref[sl, :] = new_acc
    out_ref[...] = new_acc


def kernel(x, Wg, Wgu, bgu, Wd, bd):
    b, t, d = x.shape
    E = Wg.shape[0]
    dh2 = Wgu.shape[1]
    dh = dh2 // 2
    n = b * t
    xf = x.reshape(n, d)

    bt = 512 if n % 512 == 0 else n
    n_tiles = n // bt
    dj = 1024 if dh % 1024 == 0 else dh
    nj = dh // dj

    body = functools.partial(_dense_moe_body, bt=bt, d=d, nj=nj, n_experts=E)
    out = pl.pallas_call(
        body,
        grid=(E, nj, n_tiles),
        in_specs=[
            pl.BlockSpec((bt, d), lambda e, j, i: (i, 0)),
            pl.BlockSpec((E, d), lambda e, j, i: (0, 0)),
            pl.BlockSpec((1, dj, d), lambda e, j, i: (e, j, 0)),
            pl.BlockSpec((1, dj, d),
                         lambda e, j, i, nj=nj: (e, nj + j, 0)),
            pl.BlockSpec((1, 1, dj), lambda e, j, i: (e, 0, j)),
            pl.BlockSpec((1, 1, dj),
                         lambda e, j, i, nj=nj: (e, 0, nj + j)),
            pl.BlockSpec((1, d, dj), lambda e, j, i: (e, 0, j)),
            pl.BlockSpec((1, 1, d), lambda e, j, i: (e, 0, 0)),
        ],
        out_specs=pl.BlockSpec((bt, d), lambda e, j, i: (i, 0)),
        out_shape=jax.ShapeDtypeStruct((n, d), jnp.float32),
        scratch_shapes=[pltpu.VMEM((n, d), jnp.float32)],
        compiler_params=pltpu.CompilerParams(
            dimension_semantics=("arbitrary", "arbitrary", "arbitrary")),
    )(xf, Wg, Wgu, Wgu, bgu.reshape(E, 1, dh2), bgu.reshape(E, 1, dh2),
      Wd, bd.reshape(E, 1, d))
    return out.reshape(b, t, d)


# speedup vs baseline: 1.3992x; 1.1439x over previous
import functools

import jax
import jax.numpy as jnp
from jax import lax
from jax.experimental import pallas as pl
from jax.experimental.pallas import tpu as pltpu
from jax.experimental.pallas import tpu_sc as plsc

N = 4096
D = 1024
E = 8
DH = 2048
DH2 = 2 * DH
S = 2 * N
B = 128
P = S + E * B
NB = P // B
NBP = 80

NTILE = 16
SW = S // NTILE
NW = 32
SWS = S // NW
TW = N // NW
CH = 32


def _gate_body(x_ref, wg_ref, ei_ref, gw_ref):
    xt = x_ref[...]
    logits = lax.dot_general(xt, wg_ref[...], (((1,), (1,)), ((), ())),
                             preferred_element_type=jnp.float32)
    idx = lax.broadcasted_iota(jnp.int32, logits.shape, 1)
    m1 = jnp.max(logits, axis=-1, keepdims=True)
    i1 = jnp.min(jnp.where(logits == m1, idx, E), axis=-1, keepdims=True)
    l2 = jnp.where(idx == i1, -jnp.inf, logits)
    m2 = jnp.max(l2, axis=-1, keepdims=True)
    i2 = jnp.min(jnp.where(l2 == m2, idx, E), axis=-1, keepdims=True)
    dexp = jnp.exp(m2 - m1)
    s = 1.0 / (1.0 + dexp)
    ei_ref[...] = jnp.concatenate([i1, i2], axis=1)
    gw_ref[...] = jnp.concatenate([s, dexp * s], axis=1)


def _gate(xf, Wg):
    bt = 512
    return pl.pallas_call(
        _gate_body,
        grid=(N // bt,),
        in_specs=[
            pl.BlockSpec((bt, D), lambda t: (t, 0)),
            pl.BlockSpec((E, D), lambda t: (0, 0)),
        ],
        out_specs=[
            pl.BlockSpec((bt, 2), lambda t: (t, 0)),
            pl.BlockSpec((bt, 2), lambda t: (t, 0)),
        ],
        out_shape=[
            jax.ShapeDtypeStruct((N, 2), jnp.int32),
            jax.ShapeDtypeStruct((N, 2), jnp.float32),
        ],
    )(xf, Wg)


def _lane():
    return lax.iota(jnp.int32, 16)


def _lg(v, idx):
    dnums = lax.GatherDimensionNumbers(
        offset_dims=(), collapsed_slice_dims=(0,), start_index_map=(0,))
    return lax.gather(v, idx[:, None], dnums, slice_sizes=(1,),
                      mode=lax.GatherScatterMode.PROMISE_IN_BOUNDS)


def _splat(v, j):
    return _lg(v, jnp.full((16,), j, jnp.int32))


def _cumsum16(v):
    lane = _lane()
    for sh in (1, 2, 4, 8):
        g = _lg(v, jnp.maximum(lane - sh, 0))
        v = v + jnp.where(lane >= sh, g, 0)
    return v


def _maxall16(v):
    lane = _lane()
    for sh in (1, 2, 4, 8):
        g = _lg(v, jnp.minimum(lane + sh, 15))
        v = jnp.maximum(v, g)
    return _splat(v, 0)


def _sort_body(ei_hbm, pos_hbm, be_hbm, ba_hbm,
               ev_ref, dst2d, meta0_ref, meta1_ref):
    k = lax.axis_index("s")
    lane = _lane()

    pltpu.sync_copy(ei_hbm, ev_ref)

    ones = jnp.ones((16,), jnp.int32)
    kw = k * (SW // 16)

    def hist_step(v, carry):
        accs = carry
        vec = ev_ref[pl.ds(v * 16, 16)]
        m_v = jnp.where(v < kw, 1, 0)
        new = []
        for e in range(E):
            acc_t, acc_b = accs[2 * e], accs[2 * e + 1]
            mi = jnp.where(vec == e, ones, 0)
            new.append(acc_t + mi)
            new.append(acc_b + mi * m_v)
        return tuple(new)
    zero = jnp.zeros((16,), jnp.int32)
    accs = lax.fori_loop(0, S // 16, hist_step, (zero,) * (2 * E))

    totals = jnp.zeros((16,), jnp.int32)
    base = jnp.zeros((16,), jnp.int32)
    for e in range(E):
        lm = jnp.where(lane == e, ones, 0)
        totals = totals + lm * _splat(_cumsum16(accs[2 * e]), 15)
        base = base + lm * _splat(_cumsum16(accs[2 * e + 1]), 15)

    _bshift = B.bit_length() - 1
    pc = ((totals + (B - 1)) >> _bshift) << _bshift
    incl = _cumsum16(pc)
    pad_off = incl - pc
    base = base + pad_off

    def slot_step(v, base):
        vec = ev_ref[pl.ds((kw + v) * 16, 16)]
        dest = jnp.zeros((16,), jnp.int32)
        for e in range(E):
            mi = jnp.where(vec == e, ones, 0)
            incl_m = _cumsum16(mi)
            base_e = _splat(base, e)
            dest = dest + mi * (base_e + incl_m - 1 - dest)
            base = base + jnp.where(lane == e, _splat(incl_m, 15), 0)
        row = v >> 3
        col = (v & 7) * 16
        dst2d[row, pl.ds(col, 16)] = dest
        return base
    lax.fori_loop(0, SW // 16, slot_step, base)

    for r in range(SW // 128):
        pltpu.sync_copy(dst2d.at[r],
                        pos_hbm.at[pl.ds(k * SW + r * 128, 128)])

    @pl.when(k == 0)
    def _():
        cum_end = pad_off + pc
        total = _splat(incl, 15)
        last_e = _maxall16(jnp.where(pc > 0, lane, 0))
        ones = jnp.ones((16,), jnp.int32)
        for g in range(NBP // 16):
            ids = lane + g * 16
            bs = ids * B
            eid = jnp.zeros((16,), jnp.int32)
            for e in range(E):
                end_e = _splat(cum_end, e)
                eid = eid + jnp.where(bs >= end_e, ones, 0)
            act = jnp.where(bs < total, ones, 0)
            bev = jnp.where(act == 1, eid, last_e)
            meta0_ref[pl.ds(g * 16, 16)] = bev
            meta1_ref[pl.ds(g * 16, 16)] = act
        pltpu.sync_copy(meta0_ref, be_hbm)
        pltpu.sync_copy(meta1_ref, ba_hbm)


def _sort(ei):
    mesh = plsc.VectorSubcoreMesh(core_axis_name="c", subcore_axis_name="s",
                                  num_cores=1)
    f = pl.kernel(
        _sort_body,
        out_type=[
            jax.ShapeDtypeStruct((S,), jnp.int32),
            jax.ShapeDtypeStruct((NBP,), jnp.int32),
            jax.ShapeDtypeStruct((NBP,), jnp.int32),
        ],
        mesh=mesh,
        scratch_types=[
            pltpu.VMEM((S,), jnp.int32),
            pltpu.VMEM((SW // 128, 128), jnp.int32),
            pltpu.VMEM((NBP,), jnp.int32),
            pltpu.VMEM((NBP,), jnp.int32),
        ],
    )
    return f(ei)


def _scatter_body(pos_hbm, xf_hbm, xs_hbm, idx2, buf, sem):
    wid = lax.axis_index("s") * 2 + lax.axis_index("c")
    sbase = wid * SWS
    tbase = (wid & (N // SWS - 1)) * SWS
    for c in range(SWS // CH):
        pltpu.sync_copy(pos_hbm.at[pl.ds(sbase + c * CH, CH)], idx2)
        pltpu.sync_copy(xf_hbm.at[pl.ds(tbase + c * CH, CH)], buf)
        pltpu.async_copy(buf, xs_hbm.at[idx2], sem).wait()


def _scatter_rows(pos, xf):
    mesh = plsc.VectorSubcoreMesh(core_axis_name="c", subcore_axis_name="s")
    f = pl.kernel(
        _scatter_body,
        out_type=jax.ShapeDtypeStruct((P, D), jnp.float32),
        mesh=mesh,
        scratch_types=[
            pltpu.VMEM((CH,), jnp.int32),
            pltpu.VMEM((CH, D), jnp.float32),
            pltpu.SemaphoreType.DMA,
        ],
    )
    return f(pos, xf)


def _ffn_body(be_ref, ba_ref, xs_ref, wgu_ref, bgu_ref, wd_ref, bd_ref,
              os_ref):
    i = pl.program_id(0)

    @pl.when(ba_ref[i] == 1)
    def _():
        xt = xs_ref[...]
        gu = lax.dot_general(xt, wgu_ref[0], (((1,), (1,)), ((), ())),
                             preferred_element_type=jnp.float32) + bgu_ref[0]
        g = gu[:, :DH]
        u = gu[:, DH:]
        h = g * jax.nn.sigmoid(g) * u
        os_ref[...] = lax.dot_general(
            h, wd_ref[0], (((1,), (1,)), ((), ())),
            preferred_element_type=jnp.float32) + bd_ref[0]


def _ffn(Xs, Wgu, bgu, Wd, bd, be, ba):
    grid_spec = pltpu.PrefetchScalarGridSpec(
        num_scalar_prefetch=2,
        grid=(NB,),
        in_specs=[
            pl.BlockSpec((B, D), lambda i, be, ba: (i, 0)),
            pl.BlockSpec((1, DH2, D), lambda i, be, ba: (be[i], 0, 0)),
            pl.BlockSpec((1, 1, DH2), lambda i, be, ba: (be[i], 0, 0)),
            pl.BlockSpec((1, D, DH), lambda i, be, ba: (be[i], 0, 0)),
            pl.BlockSpec((1, 1, D), lambda i, be, ba: (be[i], 0, 0)),
        ],
        out_specs=pl.BlockSpec((B, D), lambda i, be, ba: (i, 0)),
    )
    return pl.pallas_call(
        _ffn_body,
        grid_spec=grid_spec,
        out_shape=jax.ShapeDtypeStruct((P, D), jnp.float32),
        compiler_params=pltpu.CompilerParams(
            dimension_semantics=("arbitrary",)),
    )(be, ba, Xs, Wgu, bgu.reshape(E, 1, DH2), Wd, bd.reshape(E, 1, D))


def _combine_body(p0_hbm, p1_hbm, w0_hbm, w1_hbm, os_hbm, out_hbm,
                  i0_ref, i1_ref, w0_ref, w1_ref, b0, b1, ob, sem0, sem1):
    wid = lax.axis_index("s") * 2 + lax.axis_index("c")
    base = wid * TW
    for c in range(TW // CH):
        pltpu.sync_copy(p0_hbm.at[pl.ds(base + c * CH, CH)], i0_ref)
        pltpu.sync_copy(p1_hbm.at[pl.ds(base + c * CH, CH)], i1_ref)
        pltpu.sync_copy(w0_hbm.at[pl.ds(base + c * CH, CH)], w0_ref)
        pltpu.sync_copy(w1_hbm.at[pl.ds(base + c * CH, CH)], w1_ref)
        cp0 = pltpu.async_copy(os_hbm.at[i0_ref], b0, sem0)
        cp1 = pltpu.async_copy(os_hbm.at[i1_ref], b1, sem1)
        cp0.wait()
        cp1.wait()

        for r in range(CH):
            w0v = _splat(w0_ref[pl.ds((r >> 4) << 4, 16)], r & 15)
            w1v = _splat(w1_ref[pl.ds((r >> 4) << 4, 16)], r & 15)

            def add_vec(v, _, r=r, w0v=w0v, w1v=w1v):
                sl = pl.ds(v * 16, 16)
                ob[r, sl] = w0v * b0[r, sl] + w1v * b1[r, sl]
                return 0
            lax.fori_loop(0, D // 16, add_vec, 0)
        pltpu.sync_copy(ob, out_hbm.at[pl.ds(base + c * CH, CH)])


def _combine(pos0, pos1, w0, w1, Os):
    mesh = plsc.VectorSubcoreMesh(core_axis_name="c", subcore_axis_name="s")
    f = pl.kernel(
        _combine_body,
        out_type=jax.ShapeDtypeStruct((N, D), jnp.float32),
        mesh=mesh,
        scratch_types=[
            pltpu.VMEM((CH,), jnp.int32),
            pltpu.VMEM((CH,), jnp.int32),
            pltpu.VMEM((CH,), jnp.float32),
            pltpu.VMEM((CH,), jnp.float32),
            pltpu.VMEM((CH, D), jnp.float32),
            pltpu.VMEM((CH, D), jnp.float32),
            pltpu.VMEM((CH, D), jnp.float32),
            pltpu.SemaphoreType.DMA,
            pltpu.SemaphoreType.DMA,
        ],
    )
    return f(pos0, pos1, w0, w1, Os)


def kernel(x, Wg, Wgu, bgu, Wd, bd):
    b, t, d = x.shape
    assert (b * t, d) == (N, D) and Wg.shape == (E, D)
    xf = x.reshape(N, D)

    ei2, gw2 = _gate(xf, Wg)
    ei = ei2.T.reshape(S)
    pos, be, ba = _sort(ei)
    Xs = _scatter_rows(pos, xf)
    Os = _ffn(Xs, Wgu, bgu, Wd, bd, be, ba)
    out = _combine(pos[:N], pos[N:], gw2[:, 0].reshape(N),
                   gw2[:, 1].reshape(N), Os)
    return out.reshape(b, t, d)


# speedup vs baseline: 2.1093x; 1.5075x over previous
import functools

import jax
import jax.numpy as jnp
from jax import lax
from jax.experimental import pallas as pl
from jax.experimental.pallas import tpu as pltpu
from jax.experimental.pallas import tpu_sc as plsc

N = 4096
D = 1024
E = 8
DH = 2048
DH2 = 2 * DH
S = 2 * N
B = 256
P = S + E * B
NB = P // B
NBP = 48

NTILE = 16
SW = S // NTILE
NW = 32
SWS = S // NW
TW = N // NW
CH = 32


def _gate_body(x_ref, wg_ref, ei_ref, gw_ref):
    xt = x_ref[...]
    logits = lax.dot_general(xt, wg_ref[...], (((1,), (1,)), ((), ())),
                             preferred_element_type=jnp.float32)
    idx = lax.broadcasted_iota(jnp.int32, logits.shape, 1)
    m1 = jnp.max(logits, axis=-1, keepdims=True)
    i1 = jnp.min(jnp.where(logits == m1, idx, E), axis=-1, keepdims=True)
    l2 = jnp.where(idx == i1, -jnp.inf, logits)
    m2 = jnp.max(l2, axis=-1, keepdims=True)
    i2 = jnp.min(jnp.where(l2 == m2, idx, E), axis=-1, keepdims=True)
    dexp = jnp.exp(m2 - m1)
    s = 1.0 / (1.0 + dexp)
    ei_ref[...] = jnp.concatenate([i1, i2], axis=1)
    gw_ref[...] = jnp.concatenate([s, dexp * s], axis=1)


def _gate(xf, Wg):
    bt = 512
    return pl.pallas_call(
        _gate_body,
        grid=(N // bt,),
        in_specs=[
            pl.BlockSpec((bt, D), lambda t: (t, 0)),
            pl.BlockSpec((E, D), lambda t: (0, 0)),
        ],
        out_specs=[
            pl.BlockSpec((bt, 2), lambda t: (t, 0)),
            pl.BlockSpec((bt, 2), lambda t: (t, 0)),
        ],
        out_shape=[
            jax.ShapeDtypeStruct((N, 2), jnp.int32),
            jax.ShapeDtypeStruct((N, 2), jnp.float32),
        ],
    )(xf, Wg)


def _lane():
    return lax.iota(jnp.int32, 16)


def _lg(v, idx):
    dnums = lax.GatherDimensionNumbers(
        offset_dims=(), collapsed_slice_dims=(0,), start_index_map=(0,))
    return lax.gather(v, idx[:, None], dnums, slice_sizes=(1,),
                      mode=lax.GatherScatterMode.PROMISE_IN_BOUNDS)


def _splat(v, j):
    return _lg(v, jnp.full((16,), j, jnp.int32))


def _cumsum16(v):
    lane = _lane()
    for sh in (1, 2, 4, 8):
        g = _lg(v, jnp.maximum(lane - sh, 0))
        v = v + jnp.where(lane >= sh, g, 0)
    return v


def _maxall16(v):
    lane = _lane()
    for sh in (1, 2, 4, 8):
        g = _lg(v, jnp.minimum(lane + sh, 15))
        v = jnp.maximum(v, g)
    return _splat(v, 0)


def _sort_body(ei_hbm, pos_hbm, be_hbm, ba_hbm,
               ev_ref, dst2d, meta0_ref, meta1_ref):
    k = lax.axis_index("s")
    lane = _lane()

    pltpu.sync_copy(ei_hbm, ev_ref)

    ones = jnp.ones((16,), jnp.int32)
    kw = k * (SW // 16)

    def hist_step(v, carry):
        accs = carry
        vec = ev_ref[pl.ds(v * 16, 16)]
        m_v = jnp.where(v < kw, 1, 0)
        new = []
        for e in range(E):
            acc_t, acc_b = accs[2 * e], accs[2 * e + 1]
            mi = jnp.where(vec == e, ones, 0)
            new.append(acc_t + mi)
            new.append(acc_b + mi * m_v)
        return tuple(new)
    zero = jnp.zeros((16,), jnp.int32)
    accs = lax.fori_loop(0, S // 16, hist_step, (zero,) * (2 * E))

    totals = jnp.zeros((16,), jnp.int32)
    base = jnp.zeros((16,), jnp.int32)
    for e in range(E):
        lm = jnp.where(lane == e, ones, 0)
        totals = totals + lm * _splat(_cumsum16(accs[2 * e]), 15)
        base = base + lm * _splat(_cumsum16(accs[2 * e + 1]), 15)

    _bshift = B.bit_length() - 1
    pc = ((totals + (B - 1)) >> _bshift) << _bshift
    incl = _cumsum16(pc)
    pad_off = incl - pc
    base = base + pad_off

    def slot_step(v, base):
        vec = ev_ref[pl.ds((kw + v) * 16, 16)]
        dest = jnp.zeros((16,), jnp.int32)
        for e in range(E):
            mi = jnp.where(vec == e, ones, 0)
            incl_m = _cumsum16(mi)
            base_e = _splat(base, e)
            dest = dest + mi * (base_e + incl_m - 1 - dest)
            base = base + jnp.where(lane == e, _splat(incl_m, 15), 0)
        row = v >> 3
        col = (v & 7) * 16
        dst2d[row, pl.ds(col, 16)] = dest
        return base
    lax.fori_loop(0, SW // 16, slot_step, base)

    for r in range(SW // 128):
        pltpu.sync_copy(dst2d.at[r],
                        pos_hbm.at[pl.ds(k * SW + r * 128, 128)])

    @pl.when(k == 0)
    def _():
        cum_end = pad_off + pc
        total = _splat(incl, 15)
        last_e = _maxall16(jnp.where(pc > 0, lane, 0))
        ones = jnp.ones((16,), jnp.int32)
        for g in range(NBP // 16):
            ids = lane + g * 16
            bs = ids * B
            eid = jnp.zeros((16,), jnp.int32)
            for e in range(E):
                end_e = _splat(cum_end, e)
                eid = eid + jnp.where(bs >= end_e, ones, 0)
            act = jnp.where(bs < total, ones, 0)
            bev = jnp.where(act == 1, eid, last_e)
            meta0_ref[pl.ds(g * 16, 16)] = bev
            meta1_ref[pl.ds(g * 16, 16)] = act
        pltpu.sync_copy(meta0_ref, be_hbm)
        pltpu.sync_copy(meta1_ref, ba_hbm)


def _sort(ei):
    mesh = plsc.VectorSubcoreMesh(core_axis_name="c", subcore_axis_name="s",
                                  num_cores=1)
    f = pl.kernel(
        _sort_body,
        out_type=[
            jax.ShapeDtypeStruct((S,), jnp.int32),
            jax.ShapeDtypeStruct((NBP,), jnp.int32),
            jax.ShapeDtypeStruct((NBP,), jnp.int32),
        ],
        mesh=mesh,
        scratch_types=[
            pltpu.VMEM((S,), jnp.int32),
            pltpu.VMEM((SW // 128, 128), jnp.int32),
            pltpu.VMEM((NBP,), jnp.int32),
            pltpu.VMEM((NBP,), jnp.int32),
        ],
    )
    return f(ei)


def _scatter_body(pos_hbm, xf_hbm, xs_hbm, idx2, buf, sem):
    wid = lax.axis_index("s") * 2 + lax.axis_index("c")
    sbase = wid * SWS
    tbase = (wid & (N // SWS - 1)) * SWS
    for c in range(SWS // CH):
        pltpu.sync_copy(pos_hbm.at[pl.ds(sbase + c * CH, CH)], idx2)
        pltpu.sync_copy(xf_hbm.at[pl.ds(tbase + c * CH, CH)], buf)
        pltpu.async_copy(buf, xs_hbm.at[idx2], sem).wait()


def _scatter_rows(pos, xf):
    mesh = plsc.VectorSubcoreMesh(core_axis_name="c", subcore_axis_name="s")
    f = pl.kernel(
        _scatter_body,
        out_type=jax.ShapeDtypeStruct((P, D), jnp.float32),
        mesh=mesh,
        scratch_types=[
            pltpu.VMEM((CH,), jnp.int32),
            pltpu.VMEM((CH, D), jnp.float32),
            pltpu.SemaphoreType.DMA,
        ],
    )
    return f(pos, xf)


def _ffn_body(be_ref, ba_ref, xs_ref, wgu_ref, bgu_ref, wd_ref, bd_ref,
              os_ref):
    i = pl.program_id(0)

    @pl.when(ba_ref[i] == 1)
    def _():
        xt = xs_ref[...]
        gu = lax.dot_general(xt, wgu_ref[0], (((1,), (1,)), ((), ())),
                             preferred_element_type=jnp.float32) + bgu_ref[0]
        g = gu[:, :DH]
        u = gu[:, DH:]
        h = g * jax.nn.sigmoid(g) * u
        os_ref[...] = lax.dot_general(
            h, wd_ref[0], (((1,), (1,)), ((), ())),
            preferred_element_type=jnp.float32) + bd_ref[0]


def _ffn(Xs, Wgu, bgu, Wd, bd, be, ba):
    grid_spec = pltpu.PrefetchScalarGridSpec(
        num_scalar_prefetch=2,
        grid=(NB,),
        in_specs=[
            pl.BlockSpec((B, D), lambda i, be, ba: (i, 0)),
            pl.BlockSpec((1, DH2, D), lambda i, be, ba: (be[i], 0, 0)),
            pl.BlockSpec((1, 1, DH2), lambda i, be, ba: (be[i], 0, 0)),
            pl.BlockSpec((1, D, DH), lambda i, be, ba: (be[i], 0, 0)),
            pl.BlockSpec((1, 1, D), lambda i, be, ba: (be[i], 0, 0)),
        ],
        out_specs=pl.BlockSpec((B, D), lambda i, be, ba: (i, 0)),
    )
    return pl.pallas_call(
        _ffn_body,
        grid_spec=grid_spec,
        out_shape=jax.ShapeDtypeStruct((P, D), jnp.float32),
        compiler_params=pltpu.CompilerParams(
            dimension_semantics=("arbitrary",),
            vmem_limit_bytes=100 * 1024 * 1024),
    )(be, ba, Xs, Wgu, bgu.reshape(E, 1, DH2), Wd, bd.reshape(E, 1, D))


def _combine_body(p0_hbm, p1_hbm, w0_hbm, w1_hbm, os_hbm, out_hbm,
                  i0_ref, i1_ref, w0_ref, w1_ref, b0, b1, ob, sem0, sem1):
    wid = lax.axis_index("s") * 2 + lax.axis_index("c")
    base = wid * TW
    for c in range(TW // CH):
        pltpu.sync_copy(p0_hbm.at[pl.ds(base + c * CH, CH)], i0_ref)
        pltpu.sync_copy(p1_hbm.at[pl.ds(base + c * CH, CH)], i1_ref)
        pltpu.sync_copy(w0_hbm.at[pl.ds(base + c * CH, CH)], w0_ref)
        pltpu.sync_copy(w1_hbm.at[pl.ds(base + c * CH, CH)], w1_ref)
        cp0 = pltpu.async_copy(os_hbm.at[i0_ref], b0, sem0)
        cp1 = pltpu.async_copy(os_hbm.at[i1_ref], b1, sem1)
        cp0.wait()
        cp1.wait()

        for r in range(CH):
            w0v = _splat(w0_ref[pl.ds((r >> 4) << 4, 16)], r & 15)
            w1v = _splat(w1_ref[pl.ds((r >> 4) << 4, 16)], r & 15)

            def add_vec(v, _, r=r, w0v=w0v, w1v=w1v):
                sl = pl.ds(v * 16, 16)
                ob[r, sl] = w0v * b0[r, sl] + w1v * b1[r, sl]
                return 0
            lax.fori_loop(0, D // 16, add_vec, 0)
        pltpu.sync_copy(ob, out_hbm.at[pl.ds(base + c * CH, CH)])


def _combine(pos0, pos1, w0, w1, Os):
    mesh = plsc.VectorSubcoreMesh(core_axis_name="c", subcore_axis_name="s")
    f = pl.kernel(
        _combine_body,
        out_type=jax.ShapeDtypeStruct((N, D), jnp.float32),
        mesh=mesh,
        scratch_types=[
            pltpu.VMEM((CH,), jnp.int32),
            pltpu.VMEM((CH,), jnp.int32),
            pltpu.VMEM((CH,), jnp.float32),
            pltpu.VMEM((CH,), jnp.float32),
            pltpu.VMEM((CH, D), jnp.float32),
            pltpu.VMEM((CH, D), jnp.float32),
            pltpu.VMEM((CH, D), jnp.float32),
            pltpu.SemaphoreType.DMA,
            pltpu.SemaphoreType.DMA,
        ],
    )
    return f(pos0, pos1, w0, w1, Os)


def kernel(x, Wg, Wgu, bgu, Wd, bd):
    b, t, d = x.shape
    assert (b * t, d) == (N, D) and Wg.shape == (E, D)
    xf = x.reshape(N, D)

    ei2, gw2 = _gate(xf, Wg)
    ei = ei2.T.reshape(S)
    pos, be, ba = _sort(ei)
    Xs = _scatter_rows(pos, xf)
    Os = _ffn(Xs, Wgu, bgu, Wd, bd, be, ba)
    out = _combine(pos[:N], pos[N:], gw2[:, 0].reshape(N),
                   gw2[:, 1].reshape(N), Os)
    return out.reshape(b, t, d)


# speedup vs baseline: 2.1154x; 1.0029x over previous
import functools

import jax
import jax.numpy as jnp
from jax import lax
from jax.experimental import pallas as pl
from jax.experimental.pallas import tpu as pltpu
from jax.experimental.pallas import tpu_sc as plsc

N = 4096
D = 1024
E = 8
DH = 2048
DH2 = 2 * DH
S = 2 * N
B = 512
P = S + E * B
NB = P // B
NBP = 32

NTILE = 16
SW = S // NTILE
NW = 32
SWS = S // NW
TW = N // NW
CH = 32


def _gate_body(x_ref, wg_ref, ei_ref, gw_ref):
    xt = x_ref[...]
    logits = lax.dot_general(xt, wg_ref[...], (((1,), (1,)), ((), ())),
                             preferred_element_type=jnp.float32)
    idx = lax.broadcasted_iota(jnp.int32, logits.shape, 1)
    m1 = jnp.max(logits, axis=-1, keepdims=True)
    i1 = jnp.min(jnp.where(logits == m1, idx, E), axis=-1, keepdims=True)
    l2 = jnp.where(idx == i1, -jnp.inf, logits)
    m2 = jnp.max(l2, axis=-1, keepdims=True)
    i2 = jnp.min(jnp.where(l2 == m2, idx, E), axis=-1, keepdims=True)
    dexp = jnp.exp(m2 - m1)
    s = 1.0 / (1.0 + dexp)
    ei_ref[...] = jnp.concatenate([i1, i2], axis=1)
    gw_ref[...] = jnp.concatenate([s, dexp * s], axis=1)


def _gate(xf, Wg):
    bt = 512
    return pl.pallas_call(
        _gate_body,
        grid=(N // bt,),
        in_specs=[
            pl.BlockSpec((bt, D), lambda t: (t, 0)),
            pl.BlockSpec((E, D), lambda t: (0, 0)),
        ],
        out_specs=[
            pl.BlockSpec((bt, 2), lambda t: (t, 0)),
            pl.BlockSpec((bt, 2), lambda t: (t, 0)),
        ],
        out_shape=[
            jax.ShapeDtypeStruct((N, 2), jnp.int32),
            jax.ShapeDtypeStruct((N, 2), jnp.float32),
        ],
    )(xf, Wg)


def _lane():
    return lax.iota(jnp.int32, 16)


def _lg(v, idx):
    dnums = lax.GatherDimensionNumbers(
        offset_dims=(), collapsed_slice_dims=(0,), start_index_map=(0,))
    return lax.gather(v, idx[:, None], dnums, slice_sizes=(1,),
                      mode=lax.GatherScatterMode.PROMISE_IN_BOUNDS)


def _splat(v, j):
    return _lg(v, jnp.full((16,), j, jnp.int32))


def _cumsum16(v):
    lane = _lane()
    for sh in (1, 2, 4, 8):
        g = _lg(v, jnp.maximum(lane - sh, 0))
        v = v + jnp.where(lane >= sh, g, 0)
    return v


def _maxall16(v):
    lane = _lane()
    for sh in (1, 2, 4, 8):
        g = _lg(v, jnp.minimum(lane + sh, 15))
        v = jnp.maximum(v, g)
    return _splat(v, 0)


def _sort_body(ei_hbm, pos_hbm, be_hbm, ba_hbm,
               ev_ref, dst2d, meta0_ref, meta1_ref):
    k = lax.axis_index("s")
    lane = _lane()

    pltpu.sync_copy(ei_hbm, ev_ref)

    ones = jnp.ones((16,), jnp.int32)
    kw = k * (SW // 16)

    def hist_step(v, carry):
        accs = carry
        vec = ev_ref[pl.ds(v * 16, 16)]
        m_v = jnp.where(v < kw, 1, 0)
        new = []
        for e in range(E):
            acc_t, acc_b = accs[2 * e], accs[2 * e + 1]
            mi = jnp.where(vec == e, ones, 0)
            new.append(acc_t + mi)
            new.append(acc_b + mi * m_v)
        return tuple(new)
    zero = jnp.zeros((16,), jnp.int32)
    accs = lax.fori_loop(0, S // 16, hist_step, (zero,) * (2 * E))

    totals = jnp.zeros((16,), jnp.int32)
    base = jnp.zeros((16,), jnp.int32)
    for e in range(E):
        lm = jnp.where(lane == e, ones, 0)
        totals = totals + lm * _splat(_cumsum16(accs[2 * e]), 15)
        base = base + lm * _splat(_cumsum16(accs[2 * e + 1]), 15)

    _bshift = B.bit_length() - 1
    pc = ((totals + (B - 1)) >> _bshift) << _bshift
    incl = _cumsum16(pc)
    pad_off = incl - pc
    base = base + pad_off

    def slot_step(v, base):
        vec = ev_ref[pl.ds((kw + v) * 16, 16)]
        dest = jnp.zeros((16,), jnp.int32)
        for e in range(E):
            mi = jnp.where(vec == e, ones, 0)
            incl_m = _cumsum16(mi)
            base_e = _splat(base, e)
            dest = dest + mi * (base_e + incl_m - 1 - dest)
            base = base + jnp.where(lane == e, _splat(incl_m, 15), 0)
        row = v >> 3
        col = (v & 7) * 16
        dst2d[row, pl.ds(col, 16)] = dest
        return base
    lax.fori_loop(0, SW // 16, slot_step, base)

    for r in range(SW // 128):
        pltpu.sync_copy(dst2d.at[r],
                        pos_hbm.at[pl.ds(k * SW + r * 128, 128)])

    @pl.when(k == 0)
    def _():
        cum_end = pad_off + pc
        total = _splat(incl, 15)
        last_e = _maxall16(jnp.where(pc > 0, lane, 0))
        ones = jnp.ones((16,), jnp.int32)
        for g in range(NBP // 16):
            ids = lane + g * 16
            bs = ids * B
            eid = jnp.zeros((16,), jnp.int32)
            for e in range(E):
                end_e = _splat(cum_end, e)
                eid = eid + jnp.where(bs >= end_e, ones, 0)
            act = jnp.where(bs < total, ones, 0)
            bev = jnp.where(act == 1, eid, last_e)
            meta0_ref[pl.ds(g * 16, 16)] = bev
            meta1_ref[pl.ds(g * 16, 16)] = act
        pltpu.sync_copy(meta0_ref, be_hbm)
        pltpu.sync_copy(meta1_ref, ba_hbm)


def _sort(ei):
    mesh = plsc.VectorSubcoreMesh(core_axis_name="c", subcore_axis_name="s",
                                  num_cores=1)
    f = pl.kernel(
        _sort_body,
        out_type=[
            jax.ShapeDtypeStruct((S,), jnp.int32),
            jax.ShapeDtypeStruct((NBP,), jnp.int32),
            jax.ShapeDtypeStruct((NBP,), jnp.int32),
        ],
        mesh=mesh,
        scratch_types=[
            pltpu.VMEM((S,), jnp.int32),
            pltpu.VMEM((SW // 128, 128), jnp.int32),
            pltpu.VMEM((NBP,), jnp.int32),
            pltpu.VMEM((NBP,), jnp.int32),
        ],
    )
    return f(ei)


def _scatter_body(pos_hbm, xf_hbm, xs_hbm, idx2, buf, sem):
    wid = lax.axis_index("s") * 2 + lax.axis_index("c")
    sbase = wid * SWS
    tbase = (wid & (N // SWS - 1)) * SWS
    for c in range(SWS // CH):
        pltpu.sync_copy(pos_hbm.at[pl.ds(sbase + c * CH, CH)], idx2)
        pltpu.sync_copy(xf_hbm.at[pl.ds(tbase + c * CH, CH)], buf)
        pltpu.async_copy(buf, xs_hbm.at[idx2], sem).wait()


def _scatter_rows(pos, xf):
    mesh = plsc.VectorSubcoreMesh(core_axis_name="c", subcore_axis_name="s")
    f = pl.kernel(
        _scatter_body,
        out_type=jax.ShapeDtypeStruct((P, D), jnp.float32),
        mesh=mesh,
        scratch_types=[
            pltpu.VMEM((CH,), jnp.int32),
            pltpu.VMEM((CH, D), jnp.float32),
            pltpu.SemaphoreType.DMA,
        ],
    )
    return f(pos, xf)


def _ffn_body(be_ref, ba_ref, xs_ref, wgu_ref, bgu_ref, wd_ref, bd_ref,
              os_ref):
    i = pl.program_id(0)

    @pl.when(ba_ref[i] == 1)
    def _():
        xt = xs_ref[...]
        gu = lax.dot_general(xt, wgu_ref[0], (((1,), (1,)), ((), ())),
                             preferred_element_type=jnp.float32) + bgu_ref[0]
        g = gu[:, :DH]
        u = gu[:, DH:]
        h = g * jax.nn.sigmoid(g) * u
        os_ref[...] = lax.dot_general(
            h, wd_ref[0], (((1,), (1,)), ((), ())),
            preferred_element_type=jnp.float32) + bd_ref[0]


def _ffn(Xs, Wgu, bgu, Wd, bd, be, ba):
    grid_spec = pltpu.PrefetchScalarGridSpec(
        num_scalar_prefetch=2,
        grid=(NB,),
        in_specs=[
            pl.BlockSpec((B, D), lambda i, be, ba: (i, 0)),
            pl.BlockSpec((1, DH2, D), lambda i, be, ba: (be[i], 0, 0)),
            pl.BlockSpec((1, 1, DH2), lambda i, be, ba: (be[i], 0, 0)),
            pl.BlockSpec((1, D, DH), lambda i, be, ba: (be[i], 0, 0)),
            pl.BlockSpec((1, 1, D), lambda i, be, ba: (be[i], 0, 0)),
        ],
        out_specs=pl.BlockSpec((B, D), lambda i, be, ba: (i, 0)),
    )
    return pl.pallas_call(
        _ffn_body,
        grid_spec=grid_spec,
        out_shape=jax.ShapeDtypeStruct((P, D), jnp.float32),
        compiler_params=pltpu.CompilerParams(
            dimension_semantics=("arbitrary",),
            vmem_limit_bytes=100 * 1024 * 1024),
    )(be, ba, Xs, Wgu, bgu.reshape(E, 1, DH2), Wd, bd.reshape(E, 1, D))


def _combine_body(p0_hbm, p1_hbm, w0_hbm, w1_hbm, os_hbm, out_hbm,
                  i0_ref, i1_ref, w0_ref, w1_ref, b0, b1, ob, sem0, sem1):
    wid = lax.axis_index("s") * 2 + lax.axis_index("c")
    base = wid * TW
    for c in range(TW // CH):
        pltpu.sync_copy(p0_hbm.at[pl.ds(base + c * CH, CH)], i0_ref)
        pltpu.sync_copy(p1_hbm.at[pl.ds(base + c * CH, CH)], i1_ref)
        pltpu.sync_copy(w0_hbm.at[pl.ds(base + c * CH, CH)], w0_ref)
        pltpu.sync_copy(w1_hbm.at[pl.ds(base + c * CH, CH)], w1_ref)
        cp0 = pltpu.async_copy(os_hbm.at[i0_ref], b0, sem0)
        cp1 = pltpu.async_copy(os_hbm.at[i1_ref], b1, sem1)
        cp0.wait()
        cp1.wait()

        for r in range(CH):
            w0v = _splat(w0_ref[pl.ds((r >> 4) << 4, 16)], r & 15)
            w1v = _splat(w1_ref[pl.ds((r >> 4) << 4, 16)], r & 15)

            def add_vec(v, _, r=r, w0v=w0v, w1v=w1v):
                sl = pl.ds(v * 16, 16)
                ob[r, sl] = w0v * b0[r, sl] + w1v * b1[r, sl]
                return 0
            lax.fori_loop(0, D // 16, add_vec, 0)
        pltpu.sync_copy(ob, out_hbm.at[pl.ds(base + c * CH, CH)])


def _combine(pos0, pos1, w0, w1, Os):
    mesh = plsc.VectorSubcoreMesh(core_axis_name="c", subcore_axis_name="s")
    f = pl.kernel(
        _combine_body,
        out_type=jax.ShapeDtypeStruct((N, D), jnp.float32),
        mesh=mesh,
        scratch_types=[
            pltpu.VMEM((CH,), jnp.int32),
            pltpu.VMEM((CH,), jnp.int32),
            pltpu.VMEM((CH,), jnp.float32),
            pltpu.VMEM((CH,), jnp.float32),
            pltpu.VMEM((CH, D), jnp.float32),
            pltpu.VMEM((CH, D), jnp.float32),
            pltpu.VMEM((CH, D), jnp.float32),
            pltpu.SemaphoreType.DMA,
            pltpu.SemaphoreType.DMA,
        ],
    )
    return f(pos0, pos1, w0, w1, Os)


def kernel(x, Wg, Wgu, bgu, Wd, bd):
    b, t, d = x.shape
    assert (b * t, d) == (N, D) and Wg.shape == (E, D)
    xf = x.reshape(N, D)

    ei2, gw2 = _gate(xf, Wg)
    ei = ei2.T.reshape(S)
    pos, be, ba = _sort(ei)
    Xs = _scatter_rows(pos, xf)
    Os = _ffn(Xs, Wgu, bgu, Wd, bd, be, ba)
    out = _combine(pos[:N], pos[N:], gw2[:, 0].reshape(N),
                   gw2[:, 1].reshape(N), Os)
    return out.reshape(b, t, d)
